# baseline (speedup 1.0000x reference)
import functools

import jax
import jax.numpy as jnp
from jax import lax
from jax.experimental import pallas as pl
from jax.experimental.pallas import tpu as pltpu
from jax.experimental.pallas import tpu_sc as plsc

_NUM_CORES = 2
_NUM_SUBCORES = 16
_NUM_WORKERS = _NUM_CORES * _NUM_SUBCORES

_BATCH_TILE = 32
_NBUF = 2
_NSTRIPS = 4


@functools.cache
def _make_sc_gather(V, D, B, idx_dtype):
    assert B % (8 * _NUM_WORKERS) == 0
    b_per_w = B // _NUM_WORKERS
    mesh = plsc.VectorSubcoreMesh(core_axis_name="c", subcore_axis_name="s")

    @functools.partial(
        pl.kernel,
        mesh=mesh,
        out_type=jax.ShapeDtypeStruct((B, D), jnp.float32),
        scratch_types=[
            pltpu.VMEM((b_per_w,), jnp.int32),
            pltpu.VMEM((b_per_w, D), jnp.float32),
            pltpu.SemaphoreType.DMA,
        ],
        compiler_params=pltpu.CompilerParams(use_tc_tiling_on_sc=False),
    )
    def gather(table_hbm, idx_hbm, out_hbm, idx_v, rows_v, sem):
        wid = lax.axis_index("s") * _NUM_CORES + lax.axis_index("c")
        base = wid * b_per_w
        pltpu.sync_copy(idx_hbm.at[pl.ds(base, b_per_w)], idx_v)
        pltpu.async_copy(table_hbm.at[idx_v], rows_v, sem).wait()
        pltpu.sync_copy(rows_v, out_hbm.at[pl.ds(base, b_per_w)])

    return gather


def _strips(V):
    w = -(-V // (_NSTRIPS * 128)) * 128
    strips = []
    off = 0
    for _ in range(_NSTRIPS):
        strips.append((off, min(w, V - off)))
        off += w
    return strips


@functools.cache
def _make_projection(B, E, V):
    nt = B // _BATCH_TILE
    strips = _strips(V)

    def body(emb_ref, wt_ref, b_ref, out_ref, buf, sems):
        i = pl.program_id(0)
        slot = lax.rem(i, _NBUF)

        def _waits(row):
            for q, (off, w) in enumerate(strips):
                pltpu.make_async_copy(
                    buf.at[slot, :, pl.ds(off, w)],
                    out_ref.at[pl.ds(row, _BATCH_TILE), pl.ds(off, w)],
                    sems.at[slot, q],
                ).wait()

        @pl.when(i >= _NBUF)
        def _():
            _waits((i - _NBUF) * _BATCH_TILE)

        buf[slot] = jnp.broadcast_to(b_ref[...], (_BATCH_TILE, V))

        for q, (off, w) in enumerate(strips):
            pltpu.make_async_copy(
                buf.at[slot, :, pl.ds(off, w)],
                out_ref.at[pl.ds(i * _BATCH_TILE, _BATCH_TILE), pl.ds(off, w)],
                sems.at[slot, q],
            ).start()

        @pl.when(i == nt - 1)
        def _():
            for s in range(_NBUF):
                step = i - lax.rem(i - s, _NBUF)
                for q, (off, w) in enumerate(strips):
                    pltpu.make_async_copy(
                        buf.at[s, :, pl.ds(off, w)],
                        out_ref.at[pl.ds(step * _BATCH_TILE, _BATCH_TILE),
                                   pl.ds(off, w)],
                        sems.at[s, q],
                    ).wait()

    return pl.pallas_call(
        body,
        grid=(nt,),
        in_specs=[
            pl.BlockSpec((_BATCH_TILE, E), lambda i: (i, 0)),
            pl.BlockSpec((E, V), lambda i: (0, 0)),
            pl.BlockSpec((1, V), lambda i: (0, 0)),
        ],
        out_specs=pl.BlockSpec(memory_space=pl.ANY),
        out_shape=jax.ShapeDtypeStruct((B, V), jnp.float32),
        scratch_shapes=[
            pltpu.VMEM((_NBUF, _BATCH_TILE, V), jnp.float32),
            pltpu.SemaphoreType.DMA((_NBUF, _NSTRIPS)),
        ],
        compiler_params=pltpu.CompilerParams(
            dimension_semantics=("arbitrary",),
            vmem_limit_bytes=100 * 1024 * 1024,
        ),
    )


def kernel(center_idx, emb_table, W, b):
    idx = center_idx.astype(jnp.int32)
    V, E = emb_table.shape
    B = idx.shape[0]
    emb = jnp.take(emb_table, idx, axis=0)
    wt = W[:16].T
    return _make_projection(B, E, V)(emb, jnp.broadcast_to(wt[:, :1], (E, V)), b.reshape(1, V))


# speedup vs baseline: 3.0007x; 3.0007x over previous
import functools

import jax
import jax.numpy as jnp
from jax import lax
from jax.experimental import pallas as pl
from jax.experimental.pallas import tpu as pltpu
from jax.experimental.pallas import tpu_sc as plsc

_NUM_CORES = 2
_NUM_SUBCORES = 16
_NUM_WORKERS = _NUM_CORES * _NUM_SUBCORES

_VOCAB_BLOCK = 4096


@functools.cache
def _make_sc_gather(V, D, B, idx_dtype):
    assert B % (8 * _NUM_WORKERS) == 0
    b_per_w = B // _NUM_WORKERS
    mesh = plsc.VectorSubcoreMesh(core_axis_name="c", subcore_axis_name="s")

    @functools.partial(
        pl.kernel,
        mesh=mesh,
        out_type=jax.ShapeDtypeStruct((B, D), jnp.float32),
        scratch_types=[
            pltpu.VMEM((b_per_w,), jnp.int32),
            pltpu.VMEM((b_per_w, D), jnp.float32),
            pltpu.SemaphoreType.DMA,
        ],
        compiler_params=pltpu.CompilerParams(use_tc_tiling_on_sc=False),
    )
    def gather(table_hbm, idx_hbm, out_hbm, idx_v, rows_v, sem):
        wid = lax.axis_index("s") * _NUM_CORES + lax.axis_index("c")
        base = wid * b_per_w
        pltpu.sync_copy(idx_hbm.at[pl.ds(base, b_per_w)], idx_v)
        pltpu.async_copy(table_hbm.at[idx_v], rows_v, sem).wait()
        pltpu.sync_copy(rows_v, out_hbm.at[pl.ds(base, b_per_w)])

    return gather


@functools.cache
def _make_projection(B, E, V):
    def body(wt_ref, b_ref, embt_ref, out_ref):
        wa = jnp.concatenate([wt_ref[...], b_ref[...]], axis=0)
        ea = jnp.concatenate(
            [embt_ref[...], jnp.ones((1, B), jnp.float32)], axis=0
        )
        out_ref[...] = lax.dot_general(
            wa,
            ea,
            dimension_numbers=(((0,), (0,)), ((), ())),
            preferred_element_type=jnp.float32,
        )

    nt = pl.cdiv(V, _VOCAB_BLOCK)
    return pl.pallas_call(
        body,
        grid=(nt,),
        in_specs=[
            pl.BlockSpec((E, _VOCAB_BLOCK), lambda i: (0, i)),
            pl.BlockSpec((1, _VOCAB_BLOCK), lambda i: (0, i)),
            pl.BlockSpec((E, B), lambda i: (0, 0)),
        ],
        out_specs=pl.BlockSpec((_VOCAB_BLOCK, B), lambda i: (i, 0)),
        out_shape=jax.ShapeDtypeStruct((V, B), jnp.float32),
        compiler_params=pltpu.CompilerParams(
            dimension_semantics=("arbitrary",),
            vmem_limit_bytes=100 * 1024 * 1024,
        ),
    )


def kernel(center_idx, emb_table, W, b):
    idx = center_idx.astype(jnp.int32)
    V, E = emb_table.shape
    B = idx.shape[0]
    emb = jnp.take(emb_table, idx, axis=0)
    out_t = _make_projection(B, E, V)(W.T, b.reshape(1, V), emb.T)
    return out_t.T


# speedup vs baseline: 3.3713x; 1.1235x over previous
import functools

import jax
import jax.numpy as jnp
from jax import lax
from jax.experimental import pallas as pl
from jax.experimental.pallas import tpu as pltpu
from jax.experimental.pallas import tpu_sc as plsc

_NUM_CORES = 2
_NUM_SUBCORES = 16
_NUM_WORKERS = _NUM_CORES * _NUM_SUBCORES

_VOCAB_BLOCK = 4096


@functools.cache
def _make_sc_gather(V, D, B):
    assert B % (8 * _NUM_WORKERS) == 0
    b_per_w = B // _NUM_WORKERS
    mesh = plsc.VectorSubcoreMesh(core_axis_name="c", subcore_axis_name="s")

    @functools.partial(
        pl.kernel,
        mesh=mesh,
        out_type=jax.ShapeDtypeStruct((D, B), jnp.float32),
        scratch_types=[
            pltpu.VMEM((b_per_w,), jnp.int32),
            pltpu.VMEM((D, b_per_w), jnp.float32),
            pltpu.SemaphoreType.DMA,
        ],
        compiler_params=pltpu.CompilerParams(use_tc_tiling_on_sc=False),
    )
    def gather(tabt_hbm, idx_hbm, out_hbm, idx_v, vals_v, sem):
        wid = lax.axis_index("s") * _NUM_CORES + lax.axis_index("c")
        base = wid * b_per_w
        pltpu.sync_copy(idx_hbm.at[pl.ds(base, b_per_w)], idx_v)
        copies = [
            pltpu.async_copy(tabt_hbm.at[k].at[idx_v], vals_v.at[k], sem)
            for k in range(D)
        ]
        for c in copies:
            c.wait()
        pltpu.sync_copy(vals_v, out_hbm.at[:, pl.ds(base, b_per_w)])

    return gather


@functools.cache
def _make_projection(B, E, V):
    def body(wt_ref, b_ref, embt_ref, out_ref):
        wa = jnp.concatenate([wt_ref[...], b_ref[...]], axis=0)
        ea = jnp.concatenate(
            [embt_ref[...], jnp.ones((1, B), jnp.float32)], axis=0
        )
        out_ref[...] = lax.dot_general(
            wa,
            ea,
            dimension_numbers=(((0,), (0,)), ((), ())),
            preferred_element_type=jnp.float32,
        )

    nt = pl.cdiv(V, _VOCAB_BLOCK)
    return pl.pallas_call(
        body,
        grid=(nt,),
        in_specs=[
            pl.BlockSpec((E, _VOCAB_BLOCK), lambda i: (0, i)),
            pl.BlockSpec((1, _VOCAB_BLOCK), lambda i: (0, i)),
            pl.BlockSpec((E, B), lambda i: (0, 0)),
        ],
        out_specs=pl.BlockSpec((_VOCAB_BLOCK, B), lambda i: (i, 0)),
        out_shape=jax.ShapeDtypeStruct((V, B), jnp.float32),
        compiler_params=pltpu.CompilerParams(
            dimension_semantics=("arbitrary",),
            vmem_limit_bytes=100 * 1024 * 1024,
        ),
    )


def kernel(center_idx, emb_table, W, b):
    idx = center_idx.astype(jnp.int32)
    V, E = emb_table.shape
    B = idx.shape[0]
    emb_t = _make_sc_gather(V, E, B)(emb_table.T, idx)
    out_t = _make_projection(B, E, V)(W.T, b.reshape(1, V), emb_t)
    return out_t.T
